# baseline (speedup 1.0000x reference)
import functools

import jax
import jax.numpy as jnp
from jax import lax
from jax.experimental import pallas as pl
from jax.experimental.pallas import tpu as pltpu
from jax.experimental.pallas import tpu_sc as plsc

N = 10000
E = 160000
D = 256
G = 32
DH = 128
NC = 2
NS = 16
CH = 128
ROWS = E // CH
NBUF = 2
RQ = -(-(-(-ROWS // NS)) // NBUF) * NBUF
ROWS_PAD = RQ * NS
NPAD = 16
ZB = (N // NS) // 8 * 8
ZREM = N - ZB * NS
BN = 2000
GRID = N // BN


def _xl_body(h_ref, wl_ref, bl_ref, xl2_ref):
    o = jnp.dot(h_ref[...], wl_ref[...],
                preferred_element_type=jnp.float32) + bl_ref[...]
    xl2_ref[0] = o[:, :DH]
    xl2_ref[1] = o[:, DH:D]


def _film_xl_body(hp_ref, agg_ref, wgb_ref, bgb_ref, wl_ref, bl_ref,
                  xl2_ref, h_ref):
    gb = jnp.dot(hp_ref[...], wgb_ref[...],
                 preferred_element_type=jnp.float32) + bgb_ref[...]
    agg = jnp.concatenate([agg_ref[0], agg_ref[1]], axis=-1)
    h = jnp.maximum(gb[:, :D] * agg + gb[:, D:], 0.0)
    h_ref[...] = h
    o = jnp.dot(h, wl_ref[...], preferred_element_type=jnp.float32) + bl_ref[...]
    xl2_ref[0] = o[:, :DH]
    xl2_ref[1] = o[:, DH:D]


def _film_pool_body(hp_ref, agg_ref, wgb_ref, bgb_ref, batch_ref, out_ref,
                    sums_scr, counts_scr):
    i = pl.program_id(0)
    gb = jnp.dot(hp_ref[...], wgb_ref[...],
                 preferred_element_type=jnp.float32) + bgb_ref[...]
    agg = jnp.concatenate([agg_ref[0], agg_ref[1]], axis=-1)
    h = jnp.maximum(gb[:, :D] * agg + gb[:, D:], 0.0)
    bvec = batch_ref[0]
    gids = lax.broadcasted_iota(jnp.int32, (G, BN), 0)
    onehot = (gids == bvec).astype(jnp.float32)
    ps = jnp.dot(onehot, h, preferred_element_type=jnp.float32)
    pc = jnp.dot(onehot, jnp.ones((BN, D), jnp.float32),
                 preferred_element_type=jnp.float32)

    @pl.when(i == 0)
    def _():
        sums_scr[...] = ps
        counts_scr[...] = pc

    @pl.when(i > 0)
    def _():
        sums_scr[...] += ps
        counts_scr[...] += pc

    @pl.when(i == pl.num_programs(0) - 1)
    def _():
        out_ref[...] = sums_scr[...] / jnp.maximum(counts_scr[...], 1.0)


_WL_SPEC = pl.BlockSpec((D, D), lambda i: (0, 0))
_BL_SPEC = pl.BlockSpec((1, D), lambda i: (0, 0))
_WGB_SPEC = pl.BlockSpec((D, 2 * D), lambda i: (0, 0))
_BGB_SPEC = pl.BlockSpec((1, 2 * D), lambda i: (0, 0))
_H_SPEC = pl.BlockSpec((BN, D), lambda i: (i, 0))
_XL2_SPEC = pl.BlockSpec((2, BN, DH), lambda i: (0, i, 0))

_xl_call = pl.pallas_call(
    _xl_body,
    grid=(GRID,),
    in_specs=[_H_SPEC, _WL_SPEC, _BL_SPEC],
    out_specs=_XL2_SPEC,
    out_shape=jax.ShapeDtypeStruct((2, N, DH), jnp.float32),
)

_film_xl_call = pl.pallas_call(
    _film_xl_body,
    grid=(GRID,),
    in_specs=[_H_SPEC, _XL2_SPEC, _WGB_SPEC, _BGB_SPEC, _WL_SPEC, _BL_SPEC],
    out_specs=[_XL2_SPEC, _H_SPEC],
    out_shape=[
        jax.ShapeDtypeStruct((2, N, DH), jnp.float32),
        jax.ShapeDtypeStruct((N, D), jnp.float32),
    ],
)

_film_pool_call = pl.pallas_call(
    _film_pool_body,
    grid=(GRID,),
    in_specs=[_H_SPEC, _XL2_SPEC, _WGB_SPEC, _BGB_SPEC,
              pl.BlockSpec((1, 1, BN), lambda i: (i, 0, 0))],
    out_specs=pl.BlockSpec((G, D), lambda i: (0, 0)),
    out_shape=jax.ShapeDtypeStruct((G, D), jnp.float32),
    scratch_shapes=[pltpu.VMEM((G, D), jnp.float32),
                    pltpu.VMEM((G, D), jnp.float32)],
)


def _edge_body(xl_hbm, srcr_hbm, dstr_hbm, zeros_hbm, out_hbm,
               sidx, didx_all, rows, acc, isem, gsem, zsem):
    c = lax.axis_index("c")
    s = lax.axis_index("s")

    pltpu.async_copy(zeros_hbm.at[pl.ds(0, ZB)], acc.at[pl.ds(s * ZB, ZB)],
                     zsem)

    @pl.when(s == NS - 1)
    def _():
        pltpu.async_copy(zeros_hbm.at[pl.ds(0, ZREM + NPAD)],
                         acc.at[pl.ds(N - ZREM, ZREM + NPAD)], zsem)

    rowbase = pl.multiple_of(s * RQ, 8)
    pltpu.sync_copy(dstr_hbm.at[pl.ds(rowbase, RQ)], didx_all)

    for b in range(NBUF):
        pltpu.async_copy(srcr_hbm.at[pl.ds(rowbase + b, 1)],
                         sidx[b], isem[b])
    pltpu.make_async_copy(srcr_hbm.at[pl.ds(rowbase, 1)],
                          sidx[0], isem[0]).wait()
    pltpu.async_copy(xl_hbm.at[c].at[sidx[0].at[0]], rows.at[0], gsem[0])

    pltpu.make_async_copy(zeros_hbm.at[pl.ds(0, ZB)],
                          acc.at[pl.ds(s * ZB, ZB)], zsem).wait()

    @pl.when(s == NS - 1)
    def _():
        pltpu.make_async_copy(zeros_hbm.at[pl.ds(0, ZREM + NPAD)],
                              acc.at[pl.ds(N - ZREM, ZREM + NPAD)],
                              zsem).wait()

    plsc.subcore_barrier()

    def body(g, carry):
        for b in range(NBUF):
            j = g * NBUF + b
            bn = (b + 1) % NBUF
            jn = j + 1

            @pl.when(jn < RQ)
            def _():
                pltpu.make_async_copy(srcr_hbm.at[pl.ds(rowbase + jn, 1)],
                                      sidx[bn], isem[bn]).wait()
                pltpu.async_copy(xl_hbm.at[c].at[sidx[bn].at[0]],
                                 rows.at[bn], gsem[bn])

            pltpu.make_async_copy(xl_hbm.at[c].at[sidx[b].at[0]],
                                  rows.at[b], gsem[b]).wait()
            pltpu.sync_copy(rows.at[b], acc.at[didx_all.at[j]], add=True)
            jn2 = j + 2

            @pl.when(jn2 < RQ)
            def _():
                pltpu.async_copy(srcr_hbm.at[pl.ds(rowbase + jn2, 1)],
                                 sidx[b], isem[b])
        return carry

    lax.fori_loop(0, RQ // NBUF, body, 0)

    plsc.subcore_barrier()

    pltpu.sync_copy(acc.at[pl.ds(s * ZB, ZB)],
                    out_hbm.at[c, pl.ds(s * ZB, ZB)])

    @pl.when(s == NS - 1)
    def _():
        pltpu.sync_copy(acc.at[pl.ds(N - ZREM, ZREM)],
                        out_hbm.at[c, pl.ds(N - ZREM, ZREM)])


@functools.cache
def _get_edge_call():
    return pl.kernel(
        _edge_body,
        out_type=jax.ShapeDtypeStruct((2, N, DH), jnp.float32),
        mesh=plsc.VectorSubcoreMesh(core_axis_name="c", subcore_axis_name="s",
                                    num_cores=NC, num_subcores=NS),
        scratch_types=[
            [pltpu.VMEM((1, CH), jnp.int32)] * NBUF,
            pltpu.VMEM((RQ, CH), jnp.int32),
            pltpu.VMEM((NBUF, CH, DH), jnp.float32),
            pltpu.VMEM_SHARED((N + NPAD, DH), jnp.float32),
            [pltpu.SemaphoreType.DMA] * NBUF,
            [pltpu.SemaphoreType.DMA] * NBUF,
            pltpu.SemaphoreType.DMA,
        ],
    )


def _edge_call(xl2, src_r, dst_r, zeros):
    return _get_edge_call()(xl2, src_r, dst_r, zeros)


def kernel(x, edge_index, batch,
           W_lin0, b_lin0, W_gam0, b_gam0, W_bet0, b_bet0,
           W_lin1, b_lin1, W_gam1, b_gam1, W_bet1, b_bet1,
           W_lin2, b_lin2, W_gam2, b_gam2, W_bet2, b_bet2):
    npad_e = ROWS_PAD * CH - E
    src = edge_index[0].astype(jnp.int32)
    dst = edge_index[1].astype(jnp.int32)
    pad_src = jnp.zeros((npad_e,), jnp.int32)
    pad_dst = N + jnp.arange(npad_e, dtype=jnp.int32) % NPAD
    src_r = jnp.concatenate([src, pad_src]).reshape(ROWS_PAD, CH)
    dst_r = jnp.concatenate([dst, pad_dst]).reshape(ROWS_PAD, CH)
    zeros = jnp.zeros((ZB, DH), jnp.float32)
    batch3 = batch.astype(jnp.int32).reshape(GRID, 1, BN)

    params = []
    for (Wl, bl, Wg, bg, Wb, bb) in (
            (W_lin0, b_lin0, W_gam0, b_gam0, W_bet0, b_bet0),
            (W_lin1, b_lin1, W_gam1, b_gam1, W_bet1, b_bet1),
            (W_lin2, b_lin2, W_gam2, b_gam2, W_bet2, b_bet2)):
        Wgb = jnp.concatenate([Wg.T, Wb.T], axis=1)
        bgb = jnp.concatenate([bg, bb]).reshape(1, 2 * D)
        params.append((Wl.T, bl.reshape(1, D), Wgb, bgb))

    xl2 = _xl_call(x, params[0][0], params[0][1])
    agg2 = _edge_call(xl2, src_r, dst_r, zeros)
    h = x
    for i in (1, 2):
        xl2, h = _film_xl_call(h, agg2, params[i - 1][2], params[i - 1][3],
                               params[i][0], params[i][1])
        agg2 = _edge_call(xl2, src_r, dst_r, zeros)
    return _film_pool_call(h, agg2, params[2][2], params[2][3], batch3)


# speedup vs baseline: 1.0072x; 1.0072x over previous
import functools

import jax
import jax.numpy as jnp
from jax import lax
from jax.experimental import pallas as pl
from jax.experimental.pallas import tpu as pltpu
from jax.experimental.pallas import tpu_sc as plsc

N = 10000
E = 160000
D = 256
G = 32
DH = 128
NC = 2
NS = 16
CH = 128
ROWS = E // CH
NBUF = 2
RQ = -(-(-(-ROWS // NS)) // NBUF) * NBUF
ROWS_PAD = RQ * NS
NPAD = 16
ZB = (N // NS) // 8 * 8
ZREM = N - ZB * NS
BN = 5000
GRID = N // BN


def _xl_body(h_ref, wl_ref, bl_ref, xl2_ref):
    o = jnp.dot(h_ref[...], wl_ref[...],
                preferred_element_type=jnp.float32) + bl_ref[...]
    xl2_ref[0] = o[:, :DH]
    xl2_ref[1] = o[:, DH:D]


def _film_xl_body(hp_ref, agg_ref, wgb_ref, bgb_ref, wl_ref, bl_ref,
                  xl2_ref, h_ref):
    gb = jnp.dot(hp_ref[...], wgb_ref[...],
                 preferred_element_type=jnp.float32) + bgb_ref[...]
    agg = jnp.concatenate([agg_ref[0], agg_ref[1]], axis=-1)
    h = jnp.maximum(gb[:, :D] * agg + gb[:, D:], 0.0)
    h_ref[...] = h
    o = jnp.dot(h, wl_ref[...], preferred_element_type=jnp.float32) + bl_ref[...]
    xl2_ref[0] = o[:, :DH]
    xl2_ref[1] = o[:, DH:D]


def _film_pool_body(hp_ref, agg_ref, wgb_ref, bgb_ref, batch_ref, out_ref,
                    sums_scr, counts_scr):
    i = pl.program_id(0)
    gb = jnp.dot(hp_ref[...], wgb_ref[...],
                 preferred_element_type=jnp.float32) + bgb_ref[...]
    agg = jnp.concatenate([agg_ref[0], agg_ref[1]], axis=-1)
    h = jnp.maximum(gb[:, :D] * agg + gb[:, D:], 0.0)
    bvec = batch_ref[0]
    gids = lax.broadcasted_iota(jnp.int32, (G, BN), 0)
    onehot = (gids == bvec).astype(jnp.float32)
    ps = jnp.dot(onehot, h, preferred_element_type=jnp.float32)
    pc = jnp.dot(onehot, jnp.ones((BN, D), jnp.float32),
                 preferred_element_type=jnp.float32)

    @pl.when(i == 0)
    def _():
        sums_scr[...] = ps
        counts_scr[...] = pc

    @pl.when(i > 0)
    def _():
        sums_scr[...] += ps
        counts_scr[...] += pc

    @pl.when(i == pl.num_programs(0) - 1)
    def _():
        out_ref[...] = sums_scr[...] / jnp.maximum(counts_scr[...], 1.0)


_WL_SPEC = pl.BlockSpec((D, D), lambda i: (0, 0))
_BL_SPEC = pl.BlockSpec((1, D), lambda i: (0, 0))
_WGB_SPEC = pl.BlockSpec((D, 2 * D), lambda i: (0, 0))
_BGB_SPEC = pl.BlockSpec((1, 2 * D), lambda i: (0, 0))
_H_SPEC = pl.BlockSpec((BN, D), lambda i: (i, 0))
_XL2_SPEC = pl.BlockSpec((2, BN, DH), lambda i: (0, i, 0))

_xl_call = pl.pallas_call(
    _xl_body,
    grid=(GRID,),
    in_specs=[_H_SPEC, _WL_SPEC, _BL_SPEC],
    out_specs=_XL2_SPEC,
    out_shape=jax.ShapeDtypeStruct((2, N, DH), jnp.float32),
)

_film_xl_call = pl.pallas_call(
    _film_xl_body,
    grid=(GRID,),
    in_specs=[_H_SPEC, _XL2_SPEC, _WGB_SPEC, _BGB_SPEC, _WL_SPEC, _BL_SPEC],
    out_specs=[_XL2_SPEC, _H_SPEC],
    out_shape=[
        jax.ShapeDtypeStruct((2, N, DH), jnp.float32),
        jax.ShapeDtypeStruct((N, D), jnp.float32),
    ],
)

_film_pool_call = pl.pallas_call(
    _film_pool_body,
    grid=(GRID,),
    in_specs=[_H_SPEC, _XL2_SPEC, _WGB_SPEC, _BGB_SPEC,
              pl.BlockSpec((1, 1, BN), lambda i: (i, 0, 0))],
    out_specs=pl.BlockSpec((G, D), lambda i: (0, 0)),
    out_shape=jax.ShapeDtypeStruct((G, D), jnp.float32),
    scratch_shapes=[pltpu.VMEM((G, D), jnp.float32),
                    pltpu.VMEM((G, D), jnp.float32)],
)


def _edge_body(xl_hbm, srcr_hbm, dstr_hbm, zeros_hbm, out_hbm,
               sidx, didx_all, rows, acc, isem, gsem, zsem):
    c = lax.axis_index("c")
    s = lax.axis_index("s")

    pltpu.async_copy(zeros_hbm.at[pl.ds(0, ZB)], acc.at[pl.ds(s * ZB, ZB)],
                     zsem)

    @pl.when(s == NS - 1)
    def _():
        pltpu.async_copy(zeros_hbm.at[pl.ds(0, ZREM + NPAD)],
                         acc.at[pl.ds(N - ZREM, ZREM + NPAD)], zsem)

    rowbase = pl.multiple_of(s * RQ, 8)
    pltpu.sync_copy(dstr_hbm.at[pl.ds(rowbase, RQ)], didx_all)

    for b in range(NBUF):
        pltpu.async_copy(srcr_hbm.at[pl.ds(rowbase + b, 1)],
                         sidx[b], isem[b])
    pltpu.make_async_copy(srcr_hbm.at[pl.ds(rowbase, 1)],
                          sidx[0], isem[0]).wait()
    pltpu.async_copy(xl_hbm.at[c].at[sidx[0].at[0]], rows.at[0], gsem[0])

    pltpu.make_async_copy(zeros_hbm.at[pl.ds(0, ZB)],
                          acc.at[pl.ds(s * ZB, ZB)], zsem).wait()

    @pl.when(s == NS - 1)
    def _():
        pltpu.make_async_copy(zeros_hbm.at[pl.ds(0, ZREM + NPAD)],
                              acc.at[pl.ds(N - ZREM, ZREM + NPAD)],
                              zsem).wait()

    plsc.subcore_barrier()

    def body(g, carry):
        for b in range(NBUF):
            j = g * NBUF + b
            bn = (b + 1) % NBUF
            jn = j + 1

            @pl.when(jn < RQ)
            def _():
                pltpu.make_async_copy(srcr_hbm.at[pl.ds(rowbase + jn, 1)],
                                      sidx[bn], isem[bn]).wait()
                pltpu.async_copy(xl_hbm.at[c].at[sidx[bn].at[0]],
                                 rows.at[bn], gsem[bn])

            pltpu.make_async_copy(xl_hbm.at[c].at[sidx[b].at[0]],
                                  rows.at[b], gsem[b]).wait()
            pltpu.sync_copy(rows.at[b], acc.at[didx_all.at[j]], add=True)
            jn2 = j + 2

            @pl.when(jn2 < RQ)
            def _():
                pltpu.async_copy(srcr_hbm.at[pl.ds(rowbase + jn2, 1)],
                                 sidx[b], isem[b])
        return carry

    lax.fori_loop(0, RQ // NBUF, body, 0)

    plsc.subcore_barrier()

    pltpu.sync_copy(acc.at[pl.ds(s * ZB, ZB)],
                    out_hbm.at[c, pl.ds(s * ZB, ZB)])

    @pl.when(s == NS - 1)
    def _():
        pltpu.sync_copy(acc.at[pl.ds(N - ZREM, ZREM)],
                        out_hbm.at[c, pl.ds(N - ZREM, ZREM)])


@functools.cache
def _get_edge_call():
    return pl.kernel(
        _edge_body,
        out_type=jax.ShapeDtypeStruct((2, N, DH), jnp.float32),
        mesh=plsc.VectorSubcoreMesh(core_axis_name="c", subcore_axis_name="s",
                                    num_cores=NC, num_subcores=NS),
        scratch_types=[
            [pltpu.VMEM((1, CH), jnp.int32)] * NBUF,
            pltpu.VMEM((RQ, CH), jnp.int32),
            pltpu.VMEM((NBUF, CH, DH), jnp.float32),
            pltpu.VMEM_SHARED((N + NPAD, DH), jnp.float32),
            [pltpu.SemaphoreType.DMA] * NBUF,
            [pltpu.SemaphoreType.DMA] * NBUF,
            pltpu.SemaphoreType.DMA,
        ],
    )


def _edge_call(xl2, src_r, dst_r, zeros):
    return _get_edge_call()(xl2, src_r, dst_r, zeros)


def kernel(x, edge_index, batch,
           W_lin0, b_lin0, W_gam0, b_gam0, W_bet0, b_bet0,
           W_lin1, b_lin1, W_gam1, b_gam1, W_bet1, b_bet1,
           W_lin2, b_lin2, W_gam2, b_gam2, W_bet2, b_bet2):
    npad_e = ROWS_PAD * CH - E
    src = edge_index[0].astype(jnp.int32)
    dst = edge_index[1].astype(jnp.int32)
    pad_src = jnp.zeros((npad_e,), jnp.int32)
    pad_dst = N + jnp.arange(npad_e, dtype=jnp.int32) % NPAD
    src_r = jnp.concatenate([src, pad_src]).reshape(ROWS_PAD, CH)
    dst_r = jnp.concatenate([dst, pad_dst]).reshape(ROWS_PAD, CH)
    zeros = jnp.zeros((ZB, DH), jnp.float32)
    batch3 = batch.astype(jnp.int32).reshape(GRID, 1, BN)

    params = []
    for (Wl, bl, Wg, bg, Wb, bb) in (
            (W_lin0, b_lin0, W_gam0, b_gam0, W_bet0, b_bet0),
            (W_lin1, b_lin1, W_gam1, b_gam1, W_bet1, b_bet1),
            (W_lin2, b_lin2, W_gam2, b_gam2, W_bet2, b_bet2)):
        Wgb = jnp.concatenate([Wg.T, Wb.T], axis=1)
        bgb = jnp.concatenate([bg, bb]).reshape(1, 2 * D)
        params.append((Wl.T, bl.reshape(1, D), Wgb, bgb))

    xl2 = _xl_call(x, params[0][0], params[0][1])
    agg2 = _edge_call(xl2, src_r, dst_r, zeros)
    h = x
    for i in (1, 2):
        xl2, h = _film_xl_call(h, agg2, params[i - 1][2], params[i - 1][3],
                               params[i][0], params[i][1])
        agg2 = _edge_call(xl2, src_r, dst_r, zeros)
    return _film_pool_call(h, agg2, params[2][2], params[2][3], batch3)
